# Initial kernel scaffold; baseline (speedup 1.0000x reference)
import jax
import jax.numpy as jnp
from jax.experimental import pallas as pl


def kernel(x, edge_index, W1, b1, W2, b2, att_w, att_b):
    raise NotImplementedError("write your pallas kernel here")


# speedup vs baseline: 17.1403x; 17.1403x over previous
import functools

import jax
import jax.numpy as jnp
from jax import lax
from jax.experimental import pallas as pl
from jax.experimental.pallas import tpu as pltpu
from jax.experimental.pallas import tpu_sc as plsc

N = 10000
E = 320000
NW = 32
NT = 16
CH = 128
CPW = 79
E_PAD = NW * CPW * CH
N_ACC = 10240
ROWS_PER_TILE = N_ACC // NT
NCOPY = ROWS_PER_TILE // CH
DUMP = 10016
DEG_W = 8


def _mesh():
    return plsc.VectorSubcoreMesh(core_axis_name="c", subcore_axis_name="s")


@functools.cache
def _prop_kernel(d):
    @functools.partial(
        pl.kernel,
        out_type=jax.ShapeDtypeStruct((2, N_ACC, d), jnp.float32),
        mesh=_mesh(),
        scratch_types=[
            pltpu.VMEM((CPW, CH), jnp.int32),
            pltpu.VMEM((CPW, CH), jnp.int32),
            pltpu.VMEM((CH, d), jnp.float32),
            pltpu.VMEM_SHARED((N_ACC, d), jnp.float32),
            pltpu.SemaphoreType.DMA,
        ],
        compiler_params=pltpu.CompilerParams(use_tc_tiling_on_sc=False),
    )
    def prop(hs, srcp, dstp, zrow, out, src_v, dst_v, rows_v, acc, sem):
        c = lax.axis_index("c")
        t = lax.axis_index("s")
        wid = c * NT + t
        pltpu.sync_copy(zrow, rows_v)
        for j in range(NCOPY):
            pltpu.sync_copy(rows_v, acc.at[pl.ds((t * NCOPY + j) * CH, CH)])
        plsc.subcore_barrier()
        pltpu.sync_copy(srcp.at[wid], src_v)
        pltpu.sync_copy(dstp.at[wid], dst_v)

        def body(j, carry):
            pltpu.async_copy(hs.at[src_v.at[j]], rows_v, sem).wait()
            pltpu.sync_copy(rows_v, acc.at[dst_v.at[j]], add=True)
            return carry

        lax.fori_loop(0, CPW, body, 0)
        plsc.subcore_barrier()
        for j in range(NCOPY):
            off = (t * NCOPY + j) * CH
            pltpu.sync_copy(acc.at[pl.ds(off, CH)], rows_v)
            pltpu.sync_copy(rows_v, out.at[c, pl.ds(off, CH)])

    return prop


@functools.cache
def _deg_kernel():
    @functools.partial(
        pl.kernel,
        out_type=jax.ShapeDtypeStruct((2, N_ACC, DEG_W), jnp.float32),
        mesh=_mesh(),
        scratch_types=[
            pltpu.VMEM((CPW, CH), jnp.int32),
            pltpu.VMEM((CH, DEG_W), jnp.float32),
            pltpu.VMEM((CH, DEG_W), jnp.float32),
            pltpu.VMEM_SHARED((N_ACC, DEG_W), jnp.float32),
        ],
        compiler_params=pltpu.CompilerParams(use_tc_tiling_on_sc=False),
    )
    def degk(dstp, ones_hbm, zrow, out, dst_v, ones_v, buf_v, acc):
        c = lax.axis_index("c")
        t = lax.axis_index("s")
        wid = c * NT + t
        pltpu.sync_copy(zrow, buf_v)
        for j in range(NCOPY):
            pltpu.sync_copy(buf_v, acc.at[pl.ds((t * NCOPY + j) * CH, CH)])
        plsc.subcore_barrier()
        pltpu.sync_copy(dstp.at[wid], dst_v)
        pltpu.sync_copy(ones_hbm, ones_v)

        def body(j, carry):
            pltpu.sync_copy(ones_v, acc.at[dst_v.at[j]], add=True)
            return carry

        lax.fori_loop(0, CPW, body, 0)
        plsc.subcore_barrier()
        for j in range(NCOPY):
            off = (t * NCOPY + j) * CH
            pltpu.sync_copy(acc.at[pl.ds(off, CH)], buf_v)
            pltpu.sync_copy(buf_v, out.at[c, pl.ds(off, CH)])

    return degk


def _tc1(x, W1, dp):
    def body(x_ref, w_ref, dp_ref, ys_ref, s8_ref):
        deg = (dp_ref[0] + dp_ref[1])[:N]
        s8 = jnp.where(deg > 0, lax.rsqrt(jnp.maximum(deg, 1e-12)), 0.0)
        s8_ref[...] = s8
        y = jnp.dot(x_ref[...], w_ref[...], preferred_element_type=jnp.float32)
        ys_ref[...] = y * s8[:, :1]

    return pl.pallas_call(
        body,
        out_shape=(
            jax.ShapeDtypeStruct((N, 64), jnp.float32),
            jax.ShapeDtypeStruct((N, DEG_W), jnp.float32),
        ),
    )(x, W1, dp)


def _tc_combine(p, s8, d):
    def body(p_ref, s8_ref, z_ref, zs_ref):
        s = s8_ref[...][:, :1]
        z = s * (p_ref[0] + p_ref[1])[:N]
        z_ref[...] = z
        zs_ref[...] = s * z

    return pl.pallas_call(
        body,
        out_shape=(
            jax.ShapeDtypeStruct((N, d), jnp.float32),
            jax.ShapeDtypeStruct((N, d), jnp.float32),
        ),
    )(p, s8)


def _tc_mid(q, z1, s8, b1, W2, aa):
    def body(q_ref, z1_ref, s8_ref, b1_ref, w2_ref, aa_ref, us_ref):
        s = s8_ref[...][:, :1]
        z2 = s * (q_ref[0] + q_ref[1])[:N]
        h = jnp.maximum(0.5 * z1_ref[...] + (0.25 * aa_ref[0]) * z2 + b1_ref[...], 0.0)
        u = jnp.dot(h, w2_ref[...], preferred_element_type=jnp.float32)
        us_ref[...] = s * u

    return pl.pallas_call(
        body,
        in_specs=[
            pl.BlockSpec(memory_space=pltpu.VMEM),
            pl.BlockSpec(memory_space=pltpu.VMEM),
            pl.BlockSpec(memory_space=pltpu.VMEM),
            pl.BlockSpec(memory_space=pltpu.VMEM),
            pl.BlockSpec(memory_space=pltpu.VMEM),
            pl.BlockSpec(memory_space=pltpu.SMEM),
        ],
        out_shape=jax.ShapeDtypeStruct((N, 16), jnp.float32),
    )(q, z1, s8, b1, W2, aa)


def _tc_final(t, v1, s8, b2, aa):
    def body(t_ref, v1_ref, s8_ref, b2_ref, aa_ref, o_ref):
        s = s8_ref[...][:, :1]
        v2 = s * (t_ref[0] + t_ref[1])[:N]
        o = 0.5 * v1_ref[...] + (0.25 * aa_ref[0]) * v2 + b2_ref[...]
        m = jnp.max(o, axis=1, keepdims=True)
        lse = jnp.log(jnp.sum(jnp.exp(o - m), axis=1, keepdims=True)) + m
        o_ref[...] = o - lse

    return pl.pallas_call(
        body,
        in_specs=[
            pl.BlockSpec(memory_space=pltpu.VMEM),
            pl.BlockSpec(memory_space=pltpu.VMEM),
            pl.BlockSpec(memory_space=pltpu.VMEM),
            pl.BlockSpec(memory_space=pltpu.VMEM),
            pl.BlockSpec(memory_space=pltpu.SMEM),
        ],
        out_shape=jax.ShapeDtypeStruct((N, 16), jnp.float32),
    )(t, v1, s8, b2, aa)


def kernel(x, edge_index, W1, b1, W2, b2, att_w, att_b):
    src = edge_index[0].astype(jnp.int32)
    dst = edge_index[1].astype(jnp.int32)
    pad = E_PAD - E
    srcp = jnp.concatenate([src, jnp.zeros((pad,), jnp.int32)]).reshape(NW, CPW, CH)
    dstp = jnp.concatenate([dst, jnp.full((pad,), DUMP, jnp.int32)]).reshape(NW, CPW, CH)

    a = jax.nn.sigmoid(jax.nn.relu(att_w[0, 0] + att_w[1, 0]) + att_b[0])
    aa = (a * a).reshape(1).astype(jnp.float32)

    zeros64 = jnp.zeros((CH, 64), jnp.float32)
    zeros16 = jnp.zeros((CH, 16), jnp.float32)
    zeros8 = jnp.zeros((CH, DEG_W), jnp.float32)
    ones8 = jnp.ones((CH, DEG_W), jnp.float32)

    dp = _deg_kernel()(dstp, ones8, zeros8)
    ys, s8 = _tc1(x, W1, dp)

    p = _prop_kernel(64)(ys, srcp, dstp, zeros64)
    z1, ys2 = _tc_combine(p, s8, 64)
    q = _prop_kernel(64)(ys2, srcp, dstp, zeros64)
    us = _tc_mid(q, z1, s8, b1.reshape(1, 64), W2, aa)

    r = _prop_kernel(16)(us, srcp, dstp, zeros16)
    v1, us2 = _tc_combine(r, s8, 16)
    t = _prop_kernel(16)(us2, srcp, dstp, zeros16)
    return _tc_final(t, v1, s8, b2.reshape(1, 16), aa)
